# baseline (speedup 1.0000x reference)
import functools

import jax
import jax.numpy as jnp
from jax import lax
from jax.experimental import pallas as pl
from jax.experimental.pallas import tpu as pltpu
from jax.experimental.pallas import tpu_sc as plsc

N = 10000
E = 320000
D_IN = 128
D_ID = 16
D_EF = 16
D_MSG = 128
D_H = 256

NC = 2
NS = 16
LANES = 16
NW = NC * NS

EPW = E // NW
KG = 80
NCHUNK_G = EPW // KG
KS = 80
NCHUNK_S = EPW // KS
ROWS_PER_TILE = 624
TAIL_ROWS = N - NS * ROWS_PER_TILE
ZR = 48


def _gather_body(pd_hbm, ps_hbm, ei_hbm, ej_hbm, pre_hbm,
                 idxi_all, idxj_all, a0, b0, a1, b1,
                 gsa0, gsb0, gsa1, gsb1, ss0, ss1):
    wid = lax.axis_index("s") * NC + lax.axis_index("c")
    base0 = wid * EPW
    pltpu.sync_copy(ei_hbm.at[pl.ds(base0, EPW)], idxi_all)
    pltpu.sync_copy(ej_hbm.at[pl.ds(base0, EPW)], idxj_all)

    sets = ((a0, b0, gsa0, gsb0, ss0), (a1, b1, gsa1, gsb1, ss1))

    def issue(setidx, it):
        a, b, gsa, gsb, _ = sets[setidx]
        off = it * KG
        pltpu.async_copy(pd_hbm.at[idxi_all.at[pl.ds(off, KG)]], a, gsa)
        pltpu.async_copy(ps_hbm.at[idxj_all.at[pl.ds(off, KG)]], b, gsb)

    def process(setidx, it, issue_next):
        a, b, gsa, gsb, ss = sets[setidx]
        oa, ob, ogsa, ogsb, oss = sets[1 - setidx]
        pltpu.make_async_copy(pd_hbm.at[idxi_all.at[pl.ds(0, KG)]], a, gsa).wait()
        pltpu.make_async_copy(ps_hbm.at[idxj_all.at[pl.ds(0, KG)]], b, gsb).wait()
        if issue_next:
            @pl.when(it >= 1)
            def _wait_other_store():
                pltpu.make_async_copy(oa, pre_hbm.at[pl.ds(base0, KG)], oss).wait()
            issue(1 - setidx, it + 1)

        def row(r, carry):
            for h in range(2):
                for cc in range(4):
                    sl = pl.ds(cc * 32, 32)
                    a[r, h, sl] = a[r, h, sl] + b[r, h, sl]
            return carry

        lax.fori_loop(0, KG, row, 0, unroll=False)
        pltpu.async_copy(a, pre_hbm.at[pl.ds(base0 + it * KG, KG)], ss)

    issue(0, 0)

    def body2(k, carry):
        process(0, 2 * k, True)
        process(1, 2 * k + 1, True)
        return carry

    lax.fori_loop(0, (NCHUNK_G - 1) // 2, body2, 0, unroll=False)
    process(0, NCHUNK_G - 1, False)
    pltpu.make_async_copy(a0, pre_hbm.at[pl.ds(base0, KG)], ss0).wait()
    pltpu.make_async_copy(a1, pre_hbm.at[pl.ds(base0, KG)], ss1).wait()


def _scatter_body(msgs_hbm, ei3_hbm, agg_hbm, idx2_v, m0, m1, z_v, acc_sh,
                  sm0, sm1):
    c = lax.axis_index("c")
    s = lax.axis_index("s")
    wid = c * NS + s
    pltpu.sync_copy(ei3_hbm.at[wid], idx2_v)
    zero = jnp.zeros((LANES,), jnp.float32)

    def zrow(r, carry):
        for cc in range(D_MSG // LANES):
            z_v[r, pl.ds(cc * LANES, LANES)] = zero
        return carry

    lax.fori_loop(0, ZR, zrow, 0, unroll=False)
    row0 = s * ROWS_PER_TILE
    for k in range(ROWS_PER_TILE // ZR):
        pltpu.sync_copy(z_v, acc_sh.at[pl.ds(row0 + k * ZR, ZR)])

    @pl.when(s == NS - 1)
    def _zero_tail():
        pltpu.sync_copy(z_v.at[pl.ds(0, TAIL_ROWS)],
                        acc_sh.at[pl.ds(NS * ROWS_PER_TILE, TAIL_ROWS)])

    plsc.subcore_barrier()

    base0 = wid * EPW
    sets = ((m0, sm0), (m1, sm1))

    def issue(setidx, it):
        m, sm = sets[setidx]
        pltpu.async_copy(msgs_hbm.at[pl.ds(base0 + it * KS, KS)], m, sm)

    def process(setidx, it, issue_next):
        m, sm = sets[setidx]
        pltpu.make_async_copy(msgs_hbm.at[pl.ds(base0, KS)], m, sm).wait()
        if issue_next:
            issue(1 - setidx, it + 1)
        pltpu.sync_copy(m, acc_sh.at[idx2_v.at[it]], add=True)

    issue(0, 0)

    def body2(k, carry):
        process(0, 2 * k, True)
        process(1, 2 * k + 1, True)
        return carry

    lax.fori_loop(0, (NCHUNK_S - 1) // 2, body2, 0, unroll=False)
    process(0, NCHUNK_S - 1, False)
    plsc.subcore_barrier()
    pltpu.sync_copy(acc_sh.at[pl.ds(row0, ROWS_PER_TILE)],
                    agg_hbm.at[c, pl.ds(row0, ROWS_PER_TILE)])

    @pl.when(s == NS - 1)
    def _copy_tail():
        pltpu.sync_copy(acc_sh.at[pl.ds(NS * ROWS_PER_TILE, TAIL_ROWS)],
                        agg_hbm.at[c, pl.ds(NS * ROWS_PER_TILE, TAIL_ROWS)])


@functools.lru_cache(maxsize=None)
def _build_sc_kernels():
    mesh = plsc.VectorSubcoreMesh(core_axis_name="c", subcore_axis_name="s",
                                  num_cores=NC, num_subcores=NS)
    gather = pl.kernel(
        _gather_body,
        out_type=jax.ShapeDtypeStruct((E, 2, D_H // 2), jnp.bfloat16),
        mesh=mesh,
        compiler_params=pltpu.CompilerParams(use_tc_tiling_on_sc=False),
        scratch_types=[
            pltpu.VMEM((EPW,), jnp.int32),
            pltpu.VMEM((EPW,), jnp.int32),
            pltpu.VMEM((KG, 2, D_H // 2), jnp.bfloat16),
            pltpu.VMEM((KG, 2, D_H // 2), jnp.bfloat16),
            pltpu.VMEM((KG, 2, D_H // 2), jnp.bfloat16),
            pltpu.VMEM((KG, 2, D_H // 2), jnp.bfloat16),
            pltpu.SemaphoreType.DMA,
            pltpu.SemaphoreType.DMA,
            pltpu.SemaphoreType.DMA,
            pltpu.SemaphoreType.DMA,
            pltpu.SemaphoreType.DMA,
            pltpu.SemaphoreType.DMA,
        ],
    )
    scatter = pl.kernel(
        _scatter_body,
        out_type=jax.ShapeDtypeStruct((NC, N, D_MSG), jnp.float32),
        mesh=mesh,
        scratch_types=[
            pltpu.VMEM((NCHUNK_S, KS), jnp.int32),
            pltpu.VMEM((KS, D_MSG), jnp.float32),
            pltpu.VMEM((KS, D_MSG), jnp.float32),
            pltpu.VMEM((ZR, D_MSG), jnp.float32),
            pltpu.VMEM_SHARED((N, D_MSG), jnp.float32),
            pltpu.SemaphoreType.DMA,
            pltpu.SemaphoreType.DMA,
        ],
    )
    return gather, scatter


def _sc_gather(pd, ps, ei, ej):
    return _build_sc_kernels()[0](pd, ps, ei, ej)


def _sc_scatter(msgs, ei):
    ei3 = ei.reshape(NW, NCHUNK_S, KS)
    return _build_sc_kernels()[1](msgs, ei3)


_NBLK = 1000
_EBLK = 1000


def _node_pre_body(x_ref, id_ref, wxi_ref, wxj_ref, widi_ref, widj_ref,
                   u1x_ref, ub1_ref, pd_ref, ps_ref, xu_ref):
    x = x_ref[...]
    idn = id_ref[...]
    f32 = jnp.float32
    pd = (jnp.dot(x, wxi_ref[...], preferred_element_type=f32)
          + jnp.dot(idn, widi_ref[...], preferred_element_type=f32)
          ).astype(jnp.bfloat16)
    ps = (jnp.dot(x, wxj_ref[...], preferred_element_type=f32)
          + jnp.dot(idn, widj_ref[...], preferred_element_type=f32)
          ).astype(jnp.bfloat16)
    pd_ref[:, 0, :] = pd[:, :D_H // 2]
    pd_ref[:, 1, :] = pd[:, D_H // 2:]
    ps_ref[:, 0, :] = ps[:, :D_H // 2]
    ps_ref[:, 1, :] = ps[:, D_H // 2:]
    xu_ref[...] = (jnp.dot(x, u1x_ref[...], preferred_element_type=f32)
                   + ub1_ref[...])


def _node_precompute(x, identifiers, wxi, wxj, widi, widj, u1x, ub1):
    grid = (N // _NBLK,)
    full = lambda shape: pl.BlockSpec(shape, lambda i: (0,) * len(shape))
    return pl.pallas_call(
        _node_pre_body,
        grid=grid,
        in_specs=[
            pl.BlockSpec((_NBLK, D_IN), lambda i: (i, 0)),
            pl.BlockSpec((_NBLK, D_ID), lambda i: (i, 0)),
            full((D_IN, D_H)), full((D_IN, D_H)),
            full((D_ID, D_H)), full((D_ID, D_H)),
            full((D_IN, D_H)), full((1, D_H)),
        ],
        out_specs=[
            pl.BlockSpec((_NBLK, 2, D_H // 2), lambda i: (i, 0, 0)),
            pl.BlockSpec((_NBLK, 2, D_H // 2), lambda i: (i, 0, 0)),
            pl.BlockSpec((_NBLK, D_H), lambda i: (i, 0)),
        ],
        out_shape=[
            jax.ShapeDtypeStruct((N, 2, D_H // 2), jnp.bfloat16),
            jax.ShapeDtypeStruct((N, 2, D_H // 2), jnp.bfloat16),
            jax.ShapeDtypeStruct((N, D_H), jnp.float32),
        ],
    )(x, identifiers, wxi, wxj, widi, widj, u1x, ub1)


def _edge_mlp_body(pre_ref, ef_ref, wef_ref, b1_ref, w2a_ref, w2b_ref,
                   b2_ref, out_ref):
    f32 = jnp.float32
    eft = (jnp.dot(ef_ref[...], wef_ref[...], preferred_element_type=f32)
           + b1_ref[...])
    hd = D_H // 2
    h0 = jnp.maximum(pre_ref[:, 0, :].astype(f32) + eft[:, :hd], 0.0)
    h1 = jnp.maximum(pre_ref[:, 1, :].astype(f32) + eft[:, hd:], 0.0)
    out_ref[...] = (jnp.dot(h0, w2a_ref[...], preferred_element_type=f32)
                    + jnp.dot(h1, w2b_ref[...], preferred_element_type=f32)
                    + b2_ref[...])


def _edge_mlp(pre, ef, wef, b1, w2, b2):
    grid = (E // _EBLK,)
    full = lambda shape: pl.BlockSpec(shape, lambda i: (0,) * len(shape))
    hd = D_H // 2
    return pl.pallas_call(
        _edge_mlp_body,
        grid=grid,
        in_specs=[
            pl.BlockSpec((_EBLK, 2, hd), lambda i: (i, 0, 0)),
            pl.BlockSpec((_EBLK, D_EF), lambda i: (i, 0)),
            full((D_EF, D_H)), full((1, D_H)),
            full((hd, D_MSG)), full((hd, D_MSG)), full((1, D_MSG)),
        ],
        out_specs=pl.BlockSpec((_EBLK, D_MSG), lambda i: (i, 0)),
        out_shape=jax.ShapeDtypeStruct((E, D_MSG), jnp.float32),
    )(pre, ef, wef, b1, w2[:hd], w2[hd:], b2)


def _update_body(xu_ref, agg_ref, u1a_ref, w2_ref, b2_ref, out_ref):
    f32 = jnp.float32
    agg = agg_ref[0] + agg_ref[1]
    h = jnp.maximum(xu_ref[...]
                    + jnp.dot(agg, u1a_ref[...], preferred_element_type=f32),
                    0.0)
    out_ref[...] = jnp.dot(h, w2_ref[...], preferred_element_type=f32) + b2_ref[...]


def _update_mlp(xu, aggp, u1a, w2, b2):
    grid = (N // _NBLK,)
    full = lambda shape: pl.BlockSpec(shape, lambda i: (0,) * len(shape))
    return pl.pallas_call(
        _update_body,
        grid=grid,
        in_specs=[
            pl.BlockSpec((_NBLK, D_H), lambda i: (i, 0)),
            pl.BlockSpec((NC, _NBLK, D_MSG), lambda i: (0, i, 0)),
            full((D_MSG, D_H)),
            full((D_H, D_MSG)), full((1, D_MSG)),
        ],
        out_specs=pl.BlockSpec((_NBLK, D_MSG), lambda i: (i, 0)),
        out_shape=jax.ShapeDtypeStruct((N, D_MSG), jnp.float32),
    )(xu, aggp, u1a, w2, b2)


def kernel(x, edge_index, identifiers, degrees, edge_features,
           msg_W1, msg_b1, msg_W2, msg_b2,
           upd_W1, upd_b1, upd_W2, upd_b2):
    ei = edge_index[1]
    ej = edge_index[0]
    wxi = msg_W1[0:D_IN]
    wxj = msg_W1[D_IN:2 * D_IN]
    widi = msg_W1[2 * D_IN:2 * D_IN + D_ID]
    widj = msg_W1[2 * D_IN + D_ID:2 * (D_IN + D_ID)]
    wef = msg_W1[2 * (D_IN + D_ID):]
    u1x = upd_W1[0:D_IN]
    u1a = upd_W1[D_IN:]
    b1 = msg_b1.reshape(1, D_H)
    b2 = msg_b2.reshape(1, D_MSG)
    ub1 = upd_b1.reshape(1, D_H)
    ub2 = upd_b2.reshape(1, D_MSG)

    pd, ps, xu = _node_precompute(x, identifiers, wxi, wxj, widi, widj, u1x, ub1)
    pre = _sc_gather(pd, ps, ei, ej)
    msgs = _edge_mlp(pre, edge_features, wef, b1, msg_W2, b2)
    aggp = _sc_scatter(msgs, ei)
    return _update_mlp(xu, aggp, u1a, upd_W2, ub2)


# speedup vs baseline: 2.2563x; 2.2563x over previous
import functools

import jax
import jax.numpy as jnp
from jax import lax
from jax.experimental import pallas as pl
from jax.experimental.pallas import tpu as pltpu
from jax.experimental.pallas import tpu_sc as plsc

N = 10000
E = 320000
D_IN = 128
D_ID = 16
D_EF = 16
D_MSG = 128
D_H = 256

NC = 2
NS = 16
LANES = 16
NW = NC * NS

EPW = E // NW
KG = 80
NCHUNK_G = EPW // KG
KS = 80
NCHUNK_S = EPW // KS
ROWS_PER_TILE = 624
TAIL_ROWS = N - NS * ROWS_PER_TILE
ZR = 48


def _gather_body(pd_hbm, ps_hbm, ei_hbm, ej_hbm, pre_hbm,
                 idxi_all, idxj_all, a0, b0, a1, b1,
                 gsa0, gsb0, gsa1, gsb1, ss0, ss1):
    wid = lax.axis_index("s") * NC + lax.axis_index("c")
    base0 = wid * EPW
    pltpu.sync_copy(ei_hbm.at[pl.ds(base0, EPW)], idxi_all)
    pltpu.sync_copy(ej_hbm.at[pl.ds(base0, EPW)], idxj_all)

    sets = ((a0, b0, gsa0, gsb0, ss0), (a1, b1, gsa1, gsb1, ss1))

    def issue(setidx, it):
        a, b, gsa, gsb, _ = sets[setidx]
        off = it * KG
        pltpu.async_copy(pd_hbm.at[idxi_all.at[pl.ds(off, KG)]], a, gsa)
        pltpu.async_copy(ps_hbm.at[idxj_all.at[pl.ds(off, KG)]], b, gsb)

    def process(setidx, it, issue_next):
        a, b, gsa, gsb, ss = sets[setidx]
        oa, ob, ogsa, ogsb, oss = sets[1 - setidx]
        pltpu.make_async_copy(pd_hbm.at[idxi_all.at[pl.ds(0, KG)]], a, gsa).wait()
        pltpu.make_async_copy(ps_hbm.at[idxj_all.at[pl.ds(0, KG)]], b, gsb).wait()
        if issue_next:
            @pl.when(it >= 1)
            def _wait_other_store():
                pltpu.make_async_copy(oa, pre_hbm.at[pl.ds(base0, KG)], oss).wait()
            issue(1 - setidx, it + 1)

        def row(r, carry):
            for cc in range(D_MSG // LANES):
                sl = pl.ds(cc * LANES, LANES)
                av = plsc.bitcast(a[r, sl], jnp.bfloat16)
                bv = plsc.bitcast(b[r, sl], jnp.bfloat16)
                a[r, sl] = plsc.bitcast(av + bv, jnp.float32)
            return carry

        lax.fori_loop(0, KG, row, 0, unroll=False)
        pltpu.async_copy(a, pre_hbm.at[pl.ds(base0 + it * KG, KG)], ss)

    issue(0, 0)

    def body2(k, carry):
        process(0, 2 * k, True)
        process(1, 2 * k + 1, True)
        return carry

    lax.fori_loop(0, (NCHUNK_G - 1) // 2, body2, 0, unroll=False)
    process(0, NCHUNK_G - 1, False)
    pltpu.make_async_copy(a0, pre_hbm.at[pl.ds(base0, KG)], ss0).wait()
    pltpu.make_async_copy(a1, pre_hbm.at[pl.ds(base0, KG)], ss1).wait()


def _scatter_body(msgs_hbm, ei3_hbm, agg_hbm, idx2_v, m0, m1, z_v, acc_sh,
                  sm0, sm1):
    c = lax.axis_index("c")
    s = lax.axis_index("s")
    wid = c * NS + s
    pltpu.sync_copy(ei3_hbm.at[wid], idx2_v)
    zero = jnp.zeros((LANES,), jnp.float32)

    def zrow(r, carry):
        for cc in range(D_MSG // LANES):
            z_v[r, pl.ds(cc * LANES, LANES)] = zero
        return carry

    lax.fori_loop(0, ZR, zrow, 0, unroll=False)
    row0 = s * ROWS_PER_TILE
    for k in range(ROWS_PER_TILE // ZR):
        pltpu.sync_copy(z_v, acc_sh.at[pl.ds(row0 + k * ZR, ZR)])

    @pl.when(s == NS - 1)
    def _zero_tail():
        pltpu.sync_copy(z_v.at[pl.ds(0, TAIL_ROWS)],
                        acc_sh.at[pl.ds(NS * ROWS_PER_TILE, TAIL_ROWS)])

    plsc.subcore_barrier()

    base0 = wid * EPW
    sets = ((m0, sm0), (m1, sm1))

    def issue(setidx, it):
        m, sm = sets[setidx]
        pltpu.async_copy(msgs_hbm.at[pl.ds(base0 + it * KS, KS)], m, sm)

    def process(setidx, it, issue_next):
        m, sm = sets[setidx]
        pltpu.make_async_copy(msgs_hbm.at[pl.ds(base0, KS)], m, sm).wait()
        if issue_next:
            issue(1 - setidx, it + 1)
        pltpu.sync_copy(m, acc_sh.at[idx2_v.at[it]], add=True)

    issue(0, 0)

    def body2(k, carry):
        process(0, 2 * k, True)
        process(1, 2 * k + 1, True)
        return carry

    lax.fori_loop(0, (NCHUNK_S - 1) // 2, body2, 0, unroll=False)
    process(0, NCHUNK_S - 1, False)
    plsc.subcore_barrier()
    pltpu.sync_copy(acc_sh.at[pl.ds(row0, ROWS_PER_TILE)],
                    agg_hbm.at[c, pl.ds(row0, ROWS_PER_TILE)])

    @pl.when(s == NS - 1)
    def _copy_tail():
        pltpu.sync_copy(acc_sh.at[pl.ds(NS * ROWS_PER_TILE, TAIL_ROWS)],
                        agg_hbm.at[c, pl.ds(NS * ROWS_PER_TILE, TAIL_ROWS)])


@functools.lru_cache(maxsize=None)
def _build_sc_kernels():
    mesh = plsc.VectorSubcoreMesh(core_axis_name="c", subcore_axis_name="s",
                                  num_cores=NC, num_subcores=NS)
    gather = pl.kernel(
        _gather_body,
        out_type=jax.ShapeDtypeStruct((E, D_H // 2), jnp.float32),
        mesh=mesh,
        compiler_params=pltpu.CompilerParams(needs_layout_passes=False),
        scratch_types=[
            pltpu.VMEM((EPW,), jnp.int32),
            pltpu.VMEM((EPW,), jnp.int32),
            pltpu.VMEM((KG, D_H // 2), jnp.float32),
            pltpu.VMEM((KG, D_H // 2), jnp.float32),
            pltpu.VMEM((KG, D_H // 2), jnp.float32),
            pltpu.VMEM((KG, D_H // 2), jnp.float32),
            pltpu.SemaphoreType.DMA,
            pltpu.SemaphoreType.DMA,
            pltpu.SemaphoreType.DMA,
            pltpu.SemaphoreType.DMA,
            pltpu.SemaphoreType.DMA,
            pltpu.SemaphoreType.DMA,
        ],
    )
    scatter = pl.kernel(
        _scatter_body,
        out_type=jax.ShapeDtypeStruct((NC, N, D_MSG), jnp.float32),
        mesh=mesh,
        scratch_types=[
            pltpu.VMEM((NCHUNK_S, KS), jnp.int32),
            pltpu.VMEM((KS, D_MSG), jnp.float32),
            pltpu.VMEM((KS, D_MSG), jnp.float32),
            pltpu.VMEM((ZR, D_MSG), jnp.float32),
            pltpu.VMEM_SHARED((N, D_MSG), jnp.float32),
            pltpu.SemaphoreType.DMA,
            pltpu.SemaphoreType.DMA,
        ],
    )
    return gather, scatter


def _sc_gather(pd, ps, ei, ej):
    return _build_sc_kernels()[0](pd, ps, ei, ej)


def _sc_scatter(msgs, ei):
    ei3 = ei.reshape(NW, NCHUNK_S, KS)
    return _build_sc_kernels()[1](msgs, ei3)


_NBLK = 1000
_EBLK = 1000


def _bf16_pack(even, odd):
    be = lax.bitcast_convert_type(
        even.astype(jnp.bfloat16).astype(jnp.float32), jnp.uint32)
    bo = lax.bitcast_convert_type(
        odd.astype(jnp.bfloat16).astype(jnp.float32), jnp.uint32)
    w = (be >> jnp.uint32(16)) | (bo & jnp.uint32(0xFFFF0000))
    return lax.bitcast_convert_type(w, jnp.float32)


def _bf16_unpack(packed):
    u = lax.bitcast_convert_type(packed, jnp.uint32)
    even = lax.bitcast_convert_type(u << jnp.uint32(16), jnp.float32)
    odd = lax.bitcast_convert_type(u & jnp.uint32(0xFFFF0000), jnp.float32)
    return even, odd


def _node_pre_body(x_ref, id_ref, wxie_ref, wxio_ref, wxje_ref, wxjo_ref,
                   widie_ref, widio_ref, widje_ref, widjo_ref,
                   u1x_ref, ub1_ref, pd_ref, ps_ref, xu_ref):
    x = x_ref[...]
    idn = id_ref[...]
    f32 = jnp.float32
    pd_e = (jnp.dot(x, wxie_ref[...], preferred_element_type=f32)
            + jnp.dot(idn, widie_ref[...], preferred_element_type=f32))
    pd_o = (jnp.dot(x, wxio_ref[...], preferred_element_type=f32)
            + jnp.dot(idn, widio_ref[...], preferred_element_type=f32))
    ps_e = (jnp.dot(x, wxje_ref[...], preferred_element_type=f32)
            + jnp.dot(idn, widje_ref[...], preferred_element_type=f32))
    ps_o = (jnp.dot(x, wxjo_ref[...], preferred_element_type=f32)
            + jnp.dot(idn, widjo_ref[...], preferred_element_type=f32))
    pd_ref[...] = _bf16_pack(pd_e, pd_o)
    ps_ref[...] = _bf16_pack(ps_e, ps_o)
    xu_ref[...] = (jnp.dot(x, u1x_ref[...], preferred_element_type=f32)
                   + ub1_ref[...])


def _node_precompute(x, identifiers, wxi, wxj, widi, widj, u1x, ub1):
    grid = (N // _NBLK,)
    full = lambda shape: pl.BlockSpec(shape, lambda i: (0,) * len(shape))
    hd = D_H // 2
    return pl.pallas_call(
        _node_pre_body,
        grid=grid,
        in_specs=[
            pl.BlockSpec((_NBLK, D_IN), lambda i: (i, 0)),
            pl.BlockSpec((_NBLK, D_ID), lambda i: (i, 0)),
            full((D_IN, hd)), full((D_IN, hd)),
            full((D_IN, hd)), full((D_IN, hd)),
            full((D_ID, hd)), full((D_ID, hd)),
            full((D_ID, hd)), full((D_ID, hd)),
            full((D_IN, D_H)), full((1, D_H)),
        ],
        out_specs=[
            pl.BlockSpec((_NBLK, hd), lambda i: (i, 0)),
            pl.BlockSpec((_NBLK, hd), lambda i: (i, 0)),
            pl.BlockSpec((_NBLK, D_H), lambda i: (i, 0)),
        ],
        out_shape=[
            jax.ShapeDtypeStruct((N, hd), jnp.float32),
            jax.ShapeDtypeStruct((N, hd), jnp.float32),
            jax.ShapeDtypeStruct((N, D_H), jnp.float32),
        ],
    )(x, identifiers,
      wxi[:, 0::2], wxi[:, 1::2], wxj[:, 0::2], wxj[:, 1::2],
      widi[:, 0::2], widi[:, 1::2], widj[:, 0::2], widj[:, 1::2],
      u1x, ub1)


def _edge_mlp_body(pre_ref, ef_ref, wefe_ref, wefo_ref, b1e_ref, b1o_ref,
                   w2e_ref, w2o_ref, b2_ref, out_ref):
    f32 = jnp.float32
    ef = ef_ref[...]
    even, odd = _bf16_unpack(pre_ref[...])
    he = jnp.maximum(
        even + jnp.dot(ef, wefe_ref[...], preferred_element_type=f32)
        + b1e_ref[...], 0.0)
    ho = jnp.maximum(
        odd + jnp.dot(ef, wefo_ref[...], preferred_element_type=f32)
        + b1o_ref[...], 0.0)
    out_ref[...] = (jnp.dot(he, w2e_ref[...], preferred_element_type=f32)
                    + jnp.dot(ho, w2o_ref[...], preferred_element_type=f32)
                    + b2_ref[...])


def _edge_mlp(pre, ef, wef, b1, w2, b2):
    grid = (E // _EBLK,)
    full = lambda shape: pl.BlockSpec(shape, lambda i: (0,) * len(shape))
    hd = D_H // 2
    return pl.pallas_call(
        _edge_mlp_body,
        grid=grid,
        in_specs=[
            pl.BlockSpec((_EBLK, hd), lambda i: (i, 0)),
            pl.BlockSpec((_EBLK, D_EF), lambda i: (i, 0)),
            full((D_EF, hd)), full((D_EF, hd)),
            full((1, hd)), full((1, hd)),
            full((hd, D_MSG)), full((hd, D_MSG)), full((1, D_MSG)),
        ],
        out_specs=pl.BlockSpec((_EBLK, D_MSG), lambda i: (i, 0)),
        out_shape=jax.ShapeDtypeStruct((E, D_MSG), jnp.float32),
    )(pre, ef, wef[:, 0::2], wef[:, 1::2], b1[:, 0::2], b1[:, 1::2],
      w2[0::2], w2[1::2], b2)


def _update_body(xu_ref, agg_ref, u1a_ref, w2_ref, b2_ref, out_ref):
    f32 = jnp.float32
    agg = agg_ref[0] + agg_ref[1]
    h = jnp.maximum(xu_ref[...]
                    + jnp.dot(agg, u1a_ref[...], preferred_element_type=f32),
                    0.0)
    out_ref[...] = jnp.dot(h, w2_ref[...], preferred_element_type=f32) + b2_ref[...]


def _update_mlp(xu, aggp, u1a, w2, b2):
    grid = (N // _NBLK,)
    full = lambda shape: pl.BlockSpec(shape, lambda i: (0,) * len(shape))
    return pl.pallas_call(
        _update_body,
        grid=grid,
        in_specs=[
            pl.BlockSpec((_NBLK, D_H), lambda i: (i, 0)),
            pl.BlockSpec((NC, _NBLK, D_MSG), lambda i: (0, i, 0)),
            full((D_MSG, D_H)),
            full((D_H, D_MSG)), full((1, D_MSG)),
        ],
        out_specs=pl.BlockSpec((_NBLK, D_MSG), lambda i: (i, 0)),
        out_shape=jax.ShapeDtypeStruct((N, D_MSG), jnp.float32),
    )(xu, aggp, u1a, w2, b2)


def kernel(x, edge_index, identifiers, degrees, edge_features,
           msg_W1, msg_b1, msg_W2, msg_b2,
           upd_W1, upd_b1, upd_W2, upd_b2):
    ei = edge_index[1]
    ej = edge_index[0]
    wxi = msg_W1[0:D_IN]
    wxj = msg_W1[D_IN:2 * D_IN]
    widi = msg_W1[2 * D_IN:2 * D_IN + D_ID]
    widj = msg_W1[2 * D_IN + D_ID:2 * (D_IN + D_ID)]
    wef = msg_W1[2 * (D_IN + D_ID):]
    u1x = upd_W1[0:D_IN]
    u1a = upd_W1[D_IN:]
    b1 = msg_b1.reshape(1, D_H)
    b2 = msg_b2.reshape(1, D_MSG)
    ub1 = upd_b1.reshape(1, D_H)
    ub2 = upd_b2.reshape(1, D_MSG)

    pd, ps, xu = _node_precompute(x, identifiers, wxi, wxj, widi, widj, u1x, ub1)
    pre = _sc_gather(pd, ps, ei, ej)
    msgs = _edge_mlp(pre, edge_features, wef, b1, msg_W2, b2)
    aggp = _sc_scatter(msgs, ei)
    return _update_mlp(xu, aggp, u1a, upd_W2, ub2)


# speedup vs baseline: 2.6034x; 1.1538x over previous
import functools

import jax
import jax.numpy as jnp
from jax import lax
from jax.experimental import pallas as pl
from jax.experimental.pallas import tpu as pltpu
from jax.experimental.pallas import tpu_sc as plsc

N = 10000
E = 320000
D_IN = 128
D_ID = 16
D_EF = 16
D_MSG = 128
D_H = 256

NC = 2
NS = 16
LANES = 16
NW = NC * NS

EPW = E // NW
KG = 80
NCHUNK_G = EPW // KG
KS = 80
NCHUNK_S = EPW // KS
ROWS_PER_TILE = 624
TAIL_ROWS = N - NS * ROWS_PER_TILE
ZR = 48


def _gather_body(pd_hbm, ps_hbm, ei_hbm, ej_hbm, pre_hbm,
                 idxi_all, idxj_all, a0, b0, a1, b1,
                 gsa0, gsb0, gsa1, gsb1, ss0, ss1):
    wid = lax.axis_index("s") * NC + lax.axis_index("c")
    base0 = wid * EPW
    pltpu.sync_copy(ei_hbm.at[pl.ds(base0, EPW)], idxi_all)
    pltpu.sync_copy(ej_hbm.at[pl.ds(base0, EPW)], idxj_all)

    sets = ((a0, b0, gsa0, gsb0, ss0), (a1, b1, gsa1, gsb1, ss1))

    def issue(setidx, it):
        a, b, gsa, gsb, _ = sets[setidx]
        off = it * KG
        pltpu.async_copy(pd_hbm.at[idxi_all.at[pl.ds(off, KG)]], a, gsa)
        pltpu.async_copy(ps_hbm.at[idxj_all.at[pl.ds(off, KG)]], b, gsb)

    def process(setidx, it, issue_next):
        a, b, gsa, gsb, ss = sets[setidx]
        oa, ob, ogsa, ogsb, oss = sets[1 - setidx]
        pltpu.make_async_copy(pd_hbm.at[idxi_all.at[pl.ds(0, KG)]], a, gsa).wait()
        pltpu.make_async_copy(ps_hbm.at[idxj_all.at[pl.ds(0, KG)]], b, gsb).wait()
        if issue_next:
            @pl.when(it >= 1)
            def _wait_other_store():
                pltpu.make_async_copy(oa, pre_hbm.at[pl.ds(base0, KG)], oss).wait()
            issue(1 - setidx, it + 1)

        def row(r, carry):
            for cc in range(D_MSG // LANES):
                sl = pl.ds(cc * LANES, LANES)
                av = plsc.bitcast(a[r, sl], jnp.bfloat16)
                bv = plsc.bitcast(b[r, sl], jnp.bfloat16)
                a[r, sl] = plsc.bitcast(av + bv, jnp.float32)
            return carry

        lax.fori_loop(0, KG, row, 0, unroll=False)
        pltpu.async_copy(a, pre_hbm.at[pl.ds(base0 + it * KG, KG)], ss)

    issue(0, 0)

    def body2(k, carry):
        process(0, 2 * k, True)
        process(1, 2 * k + 1, True)
        return carry

    lax.fori_loop(0, (NCHUNK_G - 1) // 2, body2, 0, unroll=False)
    process(0, NCHUNK_G - 1, False)
    pltpu.make_async_copy(a0, pre_hbm.at[pl.ds(base0, KG)], ss0).wait()
    pltpu.make_async_copy(a1, pre_hbm.at[pl.ds(base0, KG)], ss1).wait()


def _scatter_body(msgs_hbm, ei3_hbm, agg_hbm, idx2_v, m0, m1, z_v, acc_sh,
                  sm0, sm1):
    c = lax.axis_index("c")
    s = lax.axis_index("s")
    wid = c * NS + s
    pltpu.sync_copy(ei3_hbm.at[wid], idx2_v)
    zero = jnp.zeros((LANES,), jnp.float32)

    def zrow(r, carry):
        for cc in range(D_MSG // LANES):
            z_v[r, pl.ds(cc * LANES, LANES)] = zero
        return carry

    lax.fori_loop(0, ZR, zrow, 0, unroll=False)
    row0 = s * ROWS_PER_TILE
    for k in range(ROWS_PER_TILE // ZR):
        pltpu.sync_copy(z_v, acc_sh.at[pl.ds(row0 + k * ZR, ZR)])

    @pl.when(s == NS - 1)
    def _zero_tail():
        pltpu.sync_copy(z_v.at[pl.ds(0, TAIL_ROWS)],
                        acc_sh.at[pl.ds(NS * ROWS_PER_TILE, TAIL_ROWS)])

    plsc.subcore_barrier()

    base0 = wid * EPW
    sets = ((m0, sm0), (m1, sm1))

    def issue(setidx, it):
        m, sm = sets[setidx]
        pltpu.async_copy(msgs_hbm.at[pl.ds(base0 + it * KS, KS)], m, sm)

    def process(setidx, it, issue_next):
        m, sm = sets[setidx]
        pltpu.make_async_copy(msgs_hbm.at[pl.ds(base0, KS)], m, sm).wait()
        if issue_next:
            issue(1 - setidx, it + 1)
        pltpu.sync_copy(m, acc_sh.at[idx2_v.at[it]], add=True)

    issue(0, 0)

    def body2(k, carry):
        process(0, 2 * k, True)
        process(1, 2 * k + 1, True)
        return carry

    lax.fori_loop(0, (NCHUNK_S - 1) // 2, body2, 0, unroll=False)
    process(0, NCHUNK_S - 1, False)
    plsc.subcore_barrier()
    pltpu.sync_copy(acc_sh.at[pl.ds(row0, ROWS_PER_TILE)],
                    agg_hbm.at[c, pl.ds(row0, ROWS_PER_TILE)])

    @pl.when(s == NS - 1)
    def _copy_tail():
        pltpu.sync_copy(acc_sh.at[pl.ds(NS * ROWS_PER_TILE, TAIL_ROWS)],
                        agg_hbm.at[c, pl.ds(NS * ROWS_PER_TILE, TAIL_ROWS)])


@functools.lru_cache(maxsize=None)
def _build_sc_kernels():
    mesh = plsc.VectorSubcoreMesh(core_axis_name="c", subcore_axis_name="s",
                                  num_cores=NC, num_subcores=NS)
    gather = pl.kernel(
        _gather_body,
        out_type=jax.ShapeDtypeStruct((E, D_H // 2), jnp.float32),
        mesh=mesh,
        compiler_params=pltpu.CompilerParams(needs_layout_passes=False),
        scratch_types=[
            pltpu.VMEM((EPW,), jnp.int32),
            pltpu.VMEM((EPW,), jnp.int32),
            pltpu.VMEM((KG, D_H // 2), jnp.float32),
            pltpu.VMEM((KG, D_H // 2), jnp.float32),
            pltpu.VMEM((KG, D_H // 2), jnp.float32),
            pltpu.VMEM((KG, D_H // 2), jnp.float32),
            pltpu.SemaphoreType.DMA,
            pltpu.SemaphoreType.DMA,
            pltpu.SemaphoreType.DMA,
            pltpu.SemaphoreType.DMA,
            pltpu.SemaphoreType.DMA,
            pltpu.SemaphoreType.DMA,
        ],
    )
    scatter = pl.kernel(
        _scatter_body,
        out_type=jax.ShapeDtypeStruct((NC, N, D_MSG), jnp.float32),
        mesh=mesh,
        scratch_types=[
            pltpu.VMEM((NCHUNK_S, KS), jnp.int32),
            pltpu.VMEM((KS, D_MSG), jnp.float32),
            pltpu.VMEM((KS, D_MSG), jnp.float32),
            pltpu.VMEM((ZR, D_MSG), jnp.float32),
            pltpu.VMEM_SHARED((N, D_MSG), jnp.float32),
            pltpu.SemaphoreType.DMA,
            pltpu.SemaphoreType.DMA,
        ],
    )
    return gather, scatter


def _sc_gather(pd, ps, ei, ej):
    return _build_sc_kernels()[0](pd, ps, ei, ej)


def _sc_scatter(msgs, ei):
    ei3 = ei.reshape(NW, NCHUNK_S, KS)
    return _build_sc_kernels()[1](msgs, ei3)


_NBLK = 2000
_EBLK = 2000


def _bf16_pack(even, odd):
    be = lax.bitcast_convert_type(
        even.astype(jnp.bfloat16).astype(jnp.float32), jnp.uint32)
    bo = lax.bitcast_convert_type(
        odd.astype(jnp.bfloat16).astype(jnp.float32), jnp.uint32)
    w = (be >> jnp.uint32(16)) | (bo & jnp.uint32(0xFFFF0000))
    return lax.bitcast_convert_type(w, jnp.float32)


def _bf16_unpack(packed):
    u = lax.bitcast_convert_type(packed, jnp.uint32)
    even = lax.bitcast_convert_type(u << jnp.uint32(16), jnp.float32)
    odd = lax.bitcast_convert_type(u & jnp.uint32(0xFFFF0000), jnp.float32)
    return even, odd


def _node_pre_body(x_ref, id_ref, wxie_ref, wxio_ref, wxje_ref, wxjo_ref,
                   widie_ref, widio_ref, widje_ref, widjo_ref,
                   u1x_ref, ub1_ref, pd_ref, ps_ref, xu_ref):
    x = x_ref[...]
    idn = id_ref[...]
    f32 = jnp.float32
    pd_e = (jnp.dot(x, wxie_ref[...], preferred_element_type=f32)
            + jnp.dot(idn, widie_ref[...], preferred_element_type=f32))
    pd_o = (jnp.dot(x, wxio_ref[...], preferred_element_type=f32)
            + jnp.dot(idn, widio_ref[...], preferred_element_type=f32))
    ps_e = (jnp.dot(x, wxje_ref[...], preferred_element_type=f32)
            + jnp.dot(idn, widje_ref[...], preferred_element_type=f32))
    ps_o = (jnp.dot(x, wxjo_ref[...], preferred_element_type=f32)
            + jnp.dot(idn, widjo_ref[...], preferred_element_type=f32))
    pd_ref[...] = _bf16_pack(pd_e, pd_o)
    ps_ref[...] = _bf16_pack(ps_e, ps_o)
    xu_ref[...] = (jnp.dot(x, u1x_ref[...], preferred_element_type=f32)
                   + ub1_ref[...])


def _node_precompute(x, identifiers, wxi, wxj, widi, widj, u1x, ub1):
    grid = (N // _NBLK,)
    full = lambda shape: pl.BlockSpec(shape, lambda i: (0,) * len(shape))
    hd = D_H // 2
    return pl.pallas_call(
        _node_pre_body,
        grid=grid,
        in_specs=[
            pl.BlockSpec((_NBLK, D_IN), lambda i: (i, 0)),
            pl.BlockSpec((_NBLK, D_ID), lambda i: (i, 0)),
            full((D_IN, hd)), full((D_IN, hd)),
            full((D_IN, hd)), full((D_IN, hd)),
            full((D_ID, hd)), full((D_ID, hd)),
            full((D_ID, hd)), full((D_ID, hd)),
            full((D_IN, D_H)), full((1, D_H)),
        ],
        out_specs=[
            pl.BlockSpec((_NBLK, hd), lambda i: (i, 0)),
            pl.BlockSpec((_NBLK, hd), lambda i: (i, 0)),
            pl.BlockSpec((_NBLK, D_H), lambda i: (i, 0)),
        ],
        out_shape=[
            jax.ShapeDtypeStruct((N, hd), jnp.float32),
            jax.ShapeDtypeStruct((N, hd), jnp.float32),
            jax.ShapeDtypeStruct((N, D_H), jnp.float32),
        ],
    )(x, identifiers,
      wxi[:, 0::2], wxi[:, 1::2], wxj[:, 0::2], wxj[:, 1::2],
      widi[:, 0::2], widi[:, 1::2], widj[:, 0::2], widj[:, 1::2],
      u1x, ub1)


def _edge_mlp_body(pre_ref, ef_ref, wefe_ref, wefo_ref, b1e_ref, b1o_ref,
                   w2e_ref, w2o_ref, b2_ref, out_ref):
    f32 = jnp.float32
    ef = ef_ref[...]
    even, odd = _bf16_unpack(pre_ref[...])
    he = jnp.maximum(
        even + jnp.dot(ef, wefe_ref[...], preferred_element_type=f32)
        + b1e_ref[...], 0.0)
    ho = jnp.maximum(
        odd + jnp.dot(ef, wefo_ref[...], preferred_element_type=f32)
        + b1o_ref[...], 0.0)
    bf = jnp.bfloat16
    out_ref[...] = (jnp.dot(he.astype(bf), w2e_ref[...].astype(bf),
                            preferred_element_type=f32)
                    + jnp.dot(ho.astype(bf), w2o_ref[...].astype(bf),
                              preferred_element_type=f32)
                    + b2_ref[...])


def _edge_mlp(pre, ef, wef, b1, w2, b2):
    grid = (E // _EBLK,)
    full = lambda shape: pl.BlockSpec(shape, lambda i: (0,) * len(shape))
    hd = D_H // 2
    return pl.pallas_call(
        _edge_mlp_body,
        grid=grid,
        in_specs=[
            pl.BlockSpec((_EBLK, hd), lambda i: (i, 0)),
            pl.BlockSpec((_EBLK, D_EF), lambda i: (i, 0)),
            full((D_EF, hd)), full((D_EF, hd)),
            full((1, hd)), full((1, hd)),
            full((hd, D_MSG)), full((hd, D_MSG)), full((1, D_MSG)),
        ],
        out_specs=pl.BlockSpec((_EBLK, D_MSG), lambda i: (i, 0)),
        out_shape=jax.ShapeDtypeStruct((E, D_MSG), jnp.float32),
    )(pre, ef, wef[:, 0::2], wef[:, 1::2], b1[:, 0::2], b1[:, 1::2],
      w2[0::2], w2[1::2], b2)


def _update_body(xu_ref, agg_ref, u1a_ref, w2_ref, b2_ref, out_ref):
    f32 = jnp.float32
    agg = agg_ref[0] + agg_ref[1]
    h = jnp.maximum(xu_ref[...]
                    + jnp.dot(agg, u1a_ref[...], preferred_element_type=f32),
                    0.0)
    out_ref[...] = jnp.dot(h, w2_ref[...], preferred_element_type=f32) + b2_ref[...]


def _update_mlp(xu, aggp, u1a, w2, b2):
    grid = (N // _NBLK,)
    full = lambda shape: pl.BlockSpec(shape, lambda i: (0,) * len(shape))
    return pl.pallas_call(
        _update_body,
        grid=grid,
        in_specs=[
            pl.BlockSpec((_NBLK, D_H), lambda i: (i, 0)),
            pl.BlockSpec((NC, _NBLK, D_MSG), lambda i: (0, i, 0)),
            full((D_MSG, D_H)),
            full((D_H, D_MSG)), full((1, D_MSG)),
        ],
        out_specs=pl.BlockSpec((_NBLK, D_MSG), lambda i: (i, 0)),
        out_shape=jax.ShapeDtypeStruct((N, D_MSG), jnp.float32),
    )(xu, aggp, u1a, w2, b2)


def kernel(x, edge_index, identifiers, degrees, edge_features,
           msg_W1, msg_b1, msg_W2, msg_b2,
           upd_W1, upd_b1, upd_W2, upd_b2):
    ei = edge_index[1]
    ej = edge_index[0]
    wxi = msg_W1[0:D_IN]
    wxj = msg_W1[D_IN:2 * D_IN]
    widi = msg_W1[2 * D_IN:2 * D_IN + D_ID]
    widj = msg_W1[2 * D_IN + D_ID:2 * (D_IN + D_ID)]
    wef = msg_W1[2 * (D_IN + D_ID):]
    u1x = upd_W1[0:D_IN]
    u1a = upd_W1[D_IN:]
    b1 = msg_b1.reshape(1, D_H)
    b2 = msg_b2.reshape(1, D_MSG)
    ub1 = upd_b1.reshape(1, D_H)
    ub2 = upd_b2.reshape(1, D_MSG)

    pd, ps, xu = _node_precompute(x, identifiers, wxi, wxj, widi, widj, u1x, ub1)
    pre = _sc_gather(pd, ps, ei, ej)
    msgs = _edge_mlp(pre, edge_features, wef, b1, msg_W2, b2)
    aggp = _sc_scatter(msgs, ei)
    return _update_mlp(xu, aggp, u1a, upd_W2, ub2)
